# baseline (speedup 1.0000x reference)
import functools

import jax
import jax.numpy as jnp
from jax import lax
from jax.experimental import pallas as pl
from jax.experimental.pallas import tpu as pltpu
from jax.experimental.pallas import tpu_sc as plsc

_K = 8192
_D = 32
_BETA = 0.25
_ROWS = 512
_PAIR = 4096
_BIG = 2**30


def _argmin_body(zb_ref, cb_ref, a_ref, b_ref, idx_ref):
    m = jax.lax.dot_general(
        zb_ref[...], cb_ref[...],
        dimension_numbers=(((1,), (1,)), ((), ())),
        preferred_element_type=jnp.float32,
    )
    d = (a_ref[...] - m) + b_ref[...]

    acc_v = None
    acc_i = None
    for p in range(_K // _PAIR):
        dp = d[:, p * _PAIR:(p + 1) * _PAIR]
        v = jnp.min(dp, axis=1, keepdims=True)
        kk = jax.lax.broadcasted_iota(jnp.int32, (_ROWS, _PAIR), 1) + p * _PAIR
        i = jnp.min(jnp.where(dp == v, kk, _BIG), axis=1, keepdims=True)
        if acc_v is None:
            acc_v, acc_i = v, i
        else:
            thr = acc_v.astype(jnp.bfloat16).astype(jnp.float32)
            upd = v < thr
            acc_v = jnp.where(upd, v, acc_v)
            acc_i = jnp.where(upd, i, acc_i)
    idx_ref[0, 0, :] = acc_i[:, 0]


def _argmin_rows(flat, codebook):
    n = flat.shape[0]
    grid = n // _ROWS
    a = jnp.sum(flat**2, axis=1, keepdims=True)
    b = jnp.sum(codebook**2, axis=1)[None, :]
    idx3 = pl.pallas_call(
        _argmin_body,
        grid=(grid,),
        in_specs=[
            pl.BlockSpec((_ROWS, _D), lambda i: (i, 0)),
            pl.BlockSpec((_K, _D), lambda i: (0, 0)),
            pl.BlockSpec((_ROWS, 1), lambda i: (i, 0)),
            pl.BlockSpec((1, _K), lambda i: (0, 0)),
        ],
        out_specs=pl.BlockSpec((1, 1, _ROWS), lambda i: (i, 0, 0)),
        out_shape=jax.ShapeDtypeStruct((grid, 1, _ROWS), jnp.int32),
        compiler_params=pltpu.CompilerParams(
            dimension_semantics=("arbitrary",),
        ),
    )((flat + flat).astype(jnp.bfloat16), codebook.astype(jnp.bfloat16), a, b)
    return idx3.reshape(n)


def _make_sc_gather(n):
    info = plsc.get_sparse_core_info()
    nc, ns = info.num_cores, info.num_subcores
    nw = nc * ns
    per_w = n // nw
    mesh = plsc.VectorSubcoreMesh(core_axis_name="c", subcore_axis_name="s")

    @functools.partial(
        pl.kernel,
        mesh=mesh,
        out_type=jax.ShapeDtypeStruct((n, _D), jnp.float32),
        scratch_types=[
            pltpu.VMEM((per_w,), jnp.int32),
            pltpu.VMEM((per_w, _D), jnp.float32),
            pltpu.SemaphoreType.DMA,
        ],
        compiler_params=pltpu.CompilerParams(use_tc_tiling_on_sc=False),
    )
    def gather_k(table_hbm, idx_hbm, out_hbm, idx_v, rows_v, sem):
        wid = lax.axis_index("s") * nc + lax.axis_index("c")
        base = wid * per_w
        pltpu.sync_copy(idx_hbm.at[pl.ds(base, per_w)], idx_v)
        pltpu.async_copy(table_hbm.at[idx_v], rows_v, sem).wait()
        pltpu.sync_copy(rows_v, out_hbm.at[pl.ds(base, per_w)])

    return gather_k


def _epilogue_body(z_ref, q_ref, qst_ref, loss_ref):
    zv = z_ref[...]
    qv = q_ref[...]
    qst_ref[...] = zv + (qv - zv)
    diff = zv - qv
    cl = jnp.sum(diff * diff) / (1.0 * zv.size)
    loss_ref[...] = jnp.reshape(cl + _BETA * cl, (1, 1))


def kernel(z, codebook):
    B, T, Dd = z.shape
    n = B * T
    flat = z.reshape(-1, Dd)
    idx = _argmin_rows(flat, codebook)
    q = _make_sc_gather(n)(codebook, idx)
    qst_flat, loss = pl.pallas_call(
        _epilogue_body,
        grid=(1,),
        in_specs=[
            pl.BlockSpec((n, Dd), lambda i: (0, 0)),
            pl.BlockSpec((n, Dd), lambda i: (0, 0)),
        ],
        out_specs=[
            pl.BlockSpec((n, Dd), lambda i: (0, 0)),
            pl.BlockSpec((1, 1), lambda i: (0, 0)),
        ],
        out_shape=[
            jax.ShapeDtypeStruct((n, Dd), jnp.float32),
            jax.ShapeDtypeStruct((1, 1), jnp.float32),
        ],
    )(flat, q)
    return qst_flat.reshape(z.shape), loss.reshape(()), idx.reshape(B, T)


# speedup vs baseline: 1.0252x; 1.0252x over previous
import functools

import jax
import jax.numpy as jnp
from jax import lax
from jax.experimental import pallas as pl
from jax.experimental.pallas import tpu as pltpu
from jax.experimental.pallas import tpu_sc as plsc

_K = 8192
_D = 32
_BETA = 0.25
_ROWS = 512
_PAIR = 4096
_BIG = 2**30


def _argmin_body(zb_ref, cb_ref, a_ref, b_ref, idx_ref):
    m = jax.lax.dot_general(
        zb_ref[...], cb_ref[...],
        dimension_numbers=(((1,), (1,)), ((), ())),
        preferred_element_type=jnp.float32,
    )
    d = (a_ref[...] - 2.0 * m) + b_ref[...]

    acc_v = None
    acc_i = None
    for p in range(_K // _PAIR):
        dp = d[:, p * _PAIR:(p + 1) * _PAIR]
        v = jnp.min(dp, axis=1, keepdims=True)
        kk = jax.lax.broadcasted_iota(jnp.int32, (_ROWS, _PAIR), 1) + p * _PAIR
        i = jnp.min(jnp.where(dp == v, kk, _BIG), axis=1, keepdims=True)
        if acc_v is None:
            acc_v, acc_i = v, i
        else:
            thr = acc_v.astype(jnp.bfloat16).astype(jnp.float32)
            upd = v < thr
            acc_v = jnp.where(upd, v, acc_v)
            acc_i = jnp.where(upd, i, acc_i)
    idx_ref[0, 0, :] = acc_i[:, 0]


def _argmin_rows(flat, codebook):
    n = flat.shape[0]
    grid = n // _ROWS
    a = jnp.sum(flat**2, axis=1, keepdims=True)
    b = jnp.sum(codebook**2, axis=1)[None, :]
    idx3 = pl.pallas_call(
        _argmin_body,
        grid=(grid,),
        in_specs=[
            pl.BlockSpec((_ROWS, _D), lambda i: (i, 0)),
            pl.BlockSpec((_K, _D), lambda i: (0, 0)),
            pl.BlockSpec((_ROWS, 1), lambda i: (i, 0)),
            pl.BlockSpec((1, _K), lambda i: (0, 0)),
        ],
        out_specs=pl.BlockSpec((1, 1, _ROWS), lambda i: (i, 0, 0)),
        out_shape=jax.ShapeDtypeStruct((grid, 1, _ROWS), jnp.int32),
        compiler_params=pltpu.CompilerParams(
            dimension_semantics=("arbitrary",),
        ),
    )(flat.astype(jnp.bfloat16), codebook.astype(jnp.bfloat16), a, b)
    return idx3.reshape(n)


def _make_sc_gather(n):
    info = plsc.get_sparse_core_info()
    nc, ns = info.num_cores, info.num_subcores
    nw = nc * ns
    per_w = n // nw
    mesh = plsc.VectorSubcoreMesh(core_axis_name="c", subcore_axis_name="s")

    @functools.partial(
        pl.kernel,
        mesh=mesh,
        out_type=jax.ShapeDtypeStruct((n, _D), jnp.float32),
        scratch_types=[
            pltpu.VMEM((per_w,), jnp.int32),
            pltpu.VMEM((per_w, _D), jnp.float32),
            pltpu.SemaphoreType.DMA,
        ],
        compiler_params=pltpu.CompilerParams(use_tc_tiling_on_sc=False),
    )
    def gather_k(table_hbm, idx_hbm, out_hbm, idx_v, rows_v, sem):
        wid = lax.axis_index("s") * nc + lax.axis_index("c")
        base = wid * per_w
        pltpu.sync_copy(idx_hbm.at[pl.ds(base, per_w)], idx_v)
        pltpu.async_copy(table_hbm.at[idx_v], rows_v, sem).wait()
        pltpu.sync_copy(rows_v, out_hbm.at[pl.ds(base, per_w)])

    return gather_k


def _epilogue_body(z_ref, q_ref, qst_ref, loss_ref):
    zv = z_ref[...]
    qv = q_ref[...]
    qst_ref[...] = zv + (qv - zv)
    diff = zv - qv
    cl = jnp.sum(diff * diff) / (1.0 * zv.size)
    loss_ref[...] = jnp.reshape(cl + _BETA * cl, (1, 1))


def kernel(z, codebook):
    B, T, Dd = z.shape
    n = B * T
    flat = z.reshape(-1, Dd)
    idx = _argmin_rows(flat, codebook)
    q = _make_sc_gather(n)(codebook, idx)
    qst_flat, loss = pl.pallas_call(
        _epilogue_body,
        grid=(1,),
        in_specs=[
            pl.BlockSpec((n, Dd), lambda i: (0, 0)),
            pl.BlockSpec((n, Dd), lambda i: (0, 0)),
        ],
        out_specs=[
            pl.BlockSpec((n, Dd), lambda i: (0, 0)),
            pl.BlockSpec((1, 1), lambda i: (0, 0)),
        ],
        out_shape=[
            jax.ShapeDtypeStruct((n, Dd), jnp.float32),
            jax.ShapeDtypeStruct((1, 1), jnp.float32),
        ],
    )(flat, q)
    return qst_flat.reshape(z.shape), loss.reshape(()), idx.reshape(B, T)
